# Initial kernel scaffold; baseline (speedup 1.0000x reference)
import jax
import jax.numpy as jnp
from jax.experimental import pallas as pl


def kernel(x, weight):
    raise NotImplementedError("write your pallas kernel here")


# speedup vs baseline: 1.0585x; 1.0585x over previous
import functools

import jax
import jax.numpy as jnp
from jax.experimental import pallas as pl

_TOPK = 8


def _router_body(x_ref, w_ref, idx_ref, wt_ref, *, n_experts):
    logits = jax.lax.dot_general(
        x_ref[...], w_ref[...],
        dimension_numbers=(((1,), (1,)), ((), ())),
        preferred_element_type=jnp.float32,
    )
    lane_ids = jax.lax.broadcasted_iota(jnp.int32, logits.shape, 1)
    vals, idxs = [], []
    l = logits
    for _ in range(_TOPK):
        m = jnp.max(l, axis=1, keepdims=True)
        idx = jnp.min(jnp.where(l >= m, lane_ids, n_experts), axis=1,
                      keepdims=True)
        vals.append(m)
        idxs.append(idx)
        l = jnp.where(lane_ids == idx, -jnp.inf, l)
    v = jnp.concatenate(vals, axis=1)
    i = jnp.concatenate(idxs, axis=1)
    e = jnp.exp(v - v[:, :1])
    wt_ref[...] = e / jnp.sum(e, axis=1, keepdims=True)
    idx_ref[...] = i


def kernel(x, weight):
    b, s, h = x.shape
    n_experts = weight.shape[0]
    t = b * s
    xt = x.reshape(t, h)
    bt = 512
    grid = (t // bt,)
    idx, wt = pl.pallas_call(
        functools.partial(_router_body, n_experts=n_experts),
        grid=grid,
        in_specs=[
            pl.BlockSpec((bt, h), lambda i: (i, 0)),
            pl.BlockSpec((n_experts, h), lambda i: (0, 0)),
        ],
        out_specs=[
            pl.BlockSpec((bt, _TOPK), lambda i: (i, 0)),
            pl.BlockSpec((bt, _TOPK), lambda i: (i, 0)),
        ],
        out_shape=[
            jax.ShapeDtypeStruct((t, _TOPK), jnp.int32),
            jax.ShapeDtypeStruct((t, _TOPK), jnp.float32),
        ],
    )(xt, weight)
    return idx, wt


# speedup vs baseline: 1.4816x; 1.3998x over previous
import functools

import jax
import jax.numpy as jnp
from jax.experimental import pallas as pl

_TOPK = 8


def _router_body(x_ref, w_ref, idx_ref, wt_ref, *, n_experts):
    logits = jax.lax.dot_general(
        w_ref[...], x_ref[...],
        dimension_numbers=(((1,), (1,)), ((), ())),
        preferred_element_type=jnp.float32,
    )
    row_ids = jax.lax.broadcasted_iota(jnp.int32, logits.shape, 0)
    vals, idxs = [], []
    l = logits
    for _ in range(_TOPK):
        m = jnp.max(l, axis=0, keepdims=True)
        idx = jnp.min(jnp.where(l >= m, row_ids, n_experts), axis=0,
                      keepdims=True)
        vals.append(m)
        idxs.append(idx)
        l = jnp.where(row_ids == idx, -jnp.inf, l)
    v = jnp.concatenate(vals, axis=0)
    i = jnp.concatenate(idxs, axis=0)
    e = jnp.exp(v - v[:1])
    wt = e / jnp.sum(e, axis=0, keepdims=True)
    idx_ref[...] = i.T
    wt_ref[...] = wt.T


def kernel(x, weight):
    b, s, h = x.shape
    n_experts = weight.shape[0]
    t = b * s
    xt = x.reshape(t, h)
    bt = 512
    grid = (t // bt,)
    idx, wt = pl.pallas_call(
        functools.partial(_router_body, n_experts=n_experts),
        grid=grid,
        in_specs=[
            pl.BlockSpec((bt, h), lambda i: (i, 0)),
            pl.BlockSpec((n_experts, h), lambda i: (0, 0)),
        ],
        out_specs=[
            pl.BlockSpec((bt, _TOPK), lambda i: (i, 0)),
            pl.BlockSpec((bt, _TOPK), lambda i: (i, 0)),
        ],
        out_shape=[
            jax.ShapeDtypeStruct((t, _TOPK), jnp.int32),
            jax.ShapeDtypeStruct((t, _TOPK), jnp.float32),
        ],
    )(xt, weight)
    return idx, wt
